# baseline (speedup 1.0000x reference)
import functools

import jax
import jax.numpy as jnp
from jax import lax
from jax.experimental import pallas as pl
from jax.experimental.pallas import tpu as pltpu
from jax.experimental.pallas import tpu_sc as plsc

B = 4096
LH = 50
D = 64
NC = 2
NS = 16
NW = NC * NS
BPW = B // NW
SLOT = 64
PACK = B * SLOT
PPW = BPW * SLOT
C = 16
ROWS = C * LH
NCHUNK = BPW // C
LANES = 16
DV = D // LANES


def _make_pack_call():
    mesh = plsc.VectorSubcoreMesh(core_axis_name="c", subcore_axis_name="s")

    @functools.partial(
        pl.kernel,
        mesh=mesh,
        out_type=jax.ShapeDtypeStruct((PACK,), jnp.int32),
        scratch_types=[
            pltpu.VMEM((BPW, LH), jnp.int32),
            pltpu.VMEM((PPW,), jnp.int32),
        ],
    )
    def sc_pack(x_hbm, xo_hbm, xblk_v, buf_v):
        wid = lax.axis_index("s") * NC + lax.axis_index("c")
        pltpu.sync_copy(x_hbm.at[pl.ds(wid * BPW, BPW)], xblk_v)
        zeros = jnp.zeros((LANES,), jnp.int32)
        for b in range(BPW):
            buf_v[pl.ds(b * SLOT + 3 * LANES, LANES)] = zeros
            for c0 in (0, LANES, 2 * LANES, 2 * LANES + 2):
                buf_v[pl.ds(b * SLOT + c0, LANES)] = xblk_v[b, pl.ds(c0, LANES)]
        pltpu.sync_copy(buf_v, xo_hbm.at[pl.ds(wid * PPW, PPW)])

    return sc_pack


def _make_embed_call():
    mesh = plsc.VectorSubcoreMesh(core_axis_name="c", subcore_axis_name="s")

    @functools.partial(
        pl.kernel,
        mesh=mesh,
        compiler_params=pltpu.CompilerParams(use_tc_tiling_on_sc=False),
        out_type=jax.ShapeDtypeStruct((B, D), jnp.float32),
        scratch_types=[
            pltpu.VMEM((PPW,), jnp.int32),
            pltpu.VMEM((ROWS, D), jnp.float32),
            pltpu.VMEM((ROWS, D), jnp.float32),
            pltpu.VMEM((C, D), jnp.float32),
            pltpu.SemaphoreType.DMA,
            pltpu.SemaphoreType.DMA,
        ],
    )
    def sc_embed(xp_hbm, tab_hbm, out_hbm, idx_v, rows0, rows1, outc_v,
                 sem0, sem1):
        wid = lax.axis_index("s") * NC + lax.axis_index("c")
        pltpu.sync_copy(xp_hbm.at[pl.ds(wid * PPW, PPW)], idx_v)

        def fire(ci, buf, sem):
            for j in range(C):
                pltpu.async_copy(
                    tab_hbm.at[idx_v.at[pl.ds((ci * C + j) * SLOT, LH)]],
                    buf.at[pl.ds(j * LH, LH)],
                    sem,
                )

        def drain(buf, sem):
            pltpu.make_async_copy(tab_hbm.at[pl.ds(0, ROWS)], buf, sem).wait()

        UNROLL = 5

        def accum_store(ci, buf):
            for b in range(C):
                def body(k, accs):
                    l = k * UNROLL
                    for u in range(UNROLL):
                        accs = tuple(
                            accs[d] + buf[b * LH + l + u,
                                          pl.ds(d * LANES, LANES)]
                            for d in range(DV)
                        )
                    return accs

                acc0 = tuple(
                    jnp.zeros((LANES,), jnp.float32) for _ in range(DV)
                )
                accs = lax.fori_loop(0, LH // UNROLL, body, acc0)
                for d in range(DV):
                    outc_v[b, pl.ds(d * LANES, LANES)] = accs[d] * (1.0 / LH)
            pltpu.sync_copy(outc_v, out_hbm.at[pl.ds(wid * BPW + ci * C, C)])

        fire(0, rows0, sem0)

        def body(i, carry):
            c0 = 2 * i
            c1 = 2 * i + 1
            fire(c1, rows1, sem1)
            drain(rows0, sem0)
            accum_store(c0, rows0)

            @pl.when(c1 + 1 < NCHUNK)
            def _():
                fire(c1 + 1, rows0, sem0)

            drain(rows1, sem1)
            accum_store(c1, rows1)
            return carry

        lax.fori_loop(0, NCHUNK // 2, body, 0)

    return sc_embed


_sc_pack = _make_pack_call()
_sc_embed = _make_embed_call()


@jax.jit
def kernel(x, table):
    xp = jnp.pad(x, ((0, 0), (0, SLOT - LH))).reshape(-1)
    return _sc_embed(xp, table)


# speedup vs baseline: 1.0070x; 1.0070x over previous
import functools

import jax
import jax.numpy as jnp
from jax import lax
from jax.experimental import pallas as pl
from jax.experimental.pallas import tpu as pltpu
from jax.experimental.pallas import tpu_sc as plsc

B = 4096
LH = 50
D = 64
NC = 2
NS = 16
NW = NC * NS
BPW = B // NW
SLOT = 64
PPW = BPW * SLOT
C = 8
ROWS = C * LH
NCHUNK = BPW // C
LANES = 16
DV = D // LANES


def _make_embed_call():
    mesh = plsc.VectorSubcoreMesh(core_axis_name="c", subcore_axis_name="s")

    @functools.partial(
        pl.kernel,
        mesh=mesh,
        compiler_params=pltpu.CompilerParams(use_tc_tiling_on_sc=False),
        out_type=jax.ShapeDtypeStruct((B, D), jnp.float32),
        scratch_types=[
            pltpu.VMEM((PPW,), jnp.int32),
            pltpu.VMEM((ROWS, D), jnp.float32),
            pltpu.VMEM((ROWS, D), jnp.float32),
            pltpu.VMEM((C, D), jnp.float32),
            pltpu.SemaphoreType.DMA,
            pltpu.SemaphoreType.DMA,
        ],
    )
    def sc_embed(xp_hbm, tab_hbm, out_hbm, idx_v, rows0, rows1, outc_v,
                 sem0, sem1):
        wid = lax.axis_index("s") * NC + lax.axis_index("c")
        pltpu.sync_copy(xp_hbm.at[pl.ds(wid * PPW, PPW)], idx_v)

        def fire(ci, buf, sem):
            for j in range(C):
                pltpu.async_copy(
                    tab_hbm.at[idx_v.at[pl.ds((ci * C + j) * SLOT, LH)]],
                    buf.at[pl.ds(j * LH, LH)],
                    sem,
                )

        def drain(buf, sem):
            pltpu.make_async_copy(tab_hbm.at[pl.ds(0, ROWS)], buf, sem).wait()

        UNROLL = 5

        def accum_store(ci, buf):
            for b in range(C):
                def body(k, accs):
                    l = k * UNROLL
                    for u in range(UNROLL):
                        accs = tuple(
                            accs[d] + buf[b * LH + l + u,
                                          pl.ds(d * LANES, LANES)]
                            for d in range(DV)
                        )
                    return accs

                acc0 = tuple(
                    jnp.zeros((LANES,), jnp.float32) for _ in range(DV)
                )
                accs = lax.fori_loop(0, LH // UNROLL, body, acc0)
                for d in range(DV):
                    outc_v[b, pl.ds(d * LANES, LANES)] = accs[d] * (1.0 / LH)
            pltpu.sync_copy(outc_v, out_hbm.at[pl.ds(wid * BPW + ci * C, C)])

        fire(0, rows0, sem0)

        def body(i, carry):
            c0 = 2 * i
            c1 = 2 * i + 1
            fire(c1, rows1, sem1)
            drain(rows0, sem0)
            accum_store(c0, rows0)

            @pl.when(c1 + 1 < NCHUNK)
            def _():
                fire(c1 + 1, rows0, sem0)

            drain(rows1, sem1)
            accum_store(c1, rows1)
            return carry

        lax.fori_loop(0, NCHUNK // 2, body, 0)

    return sc_embed


_sc_embed = _make_embed_call()


@jax.jit
def kernel(x, table):
    xp = jnp.pad(x, ((0, 0), (0, SLOT - LH))).reshape(-1)
    return _sc_embed(xp, table)
